# baseline (speedup 1.0000x reference)
import functools

import jax
import jax.numpy as jnp
from jax import lax
from jax.experimental import pallas as pl
from jax.experimental.pallas import tpu as pltpu
from jax.experimental.pallas import tpu_sc as plsc

N = 10000
NP = 10240
CH = 128
OUT = 64
E_RAW = 320000
E_TOT = E_RAW + N
NC = 2
NS = 16
NW = NC * NS
BLK = 112
NB0 = 72
NB1 = 114
TOTB = NS * (NB0 + NB1)
EP = TOTB * BLK
PAD_IDX = N
RPT = NP // NS
NRING = 3
IRING = 8


def _tc_pre_body(x_ref, w_ref, asr_ref, adr_ref, h_ref, as_ref, ad_ref, m_ref):
    h = jnp.dot(x_ref[...], w_ref[...], preferred_element_type=jnp.float32)
    h_ref[...] = h
    a_s = jnp.sum(h * asr_ref[...][None, :], axis=1)
    a_d = jnp.sum(h * adr_ref[...][None, :], axis=1)
    as_ref[...] = a_s
    ad_ref[...] = a_d
    t = jnp.max(a_s) + a_d
    m_ref[...] = jnp.where(t > 0, t, 0.2 * t)


def _tc_mid_body(op_ref, dp_ref, b_ref, w_ref, asr_ref, adr_ref,
                 h_ref, as_ref, ad_ref, m_ref):
    den = dp_ref[0, :] + dp_ref[1, :] + 1e-16
    o = (op_ref[0] + op_ref[1]) / den[:, None] + b_ref[...][None, :]
    o = jnp.maximum(o, 0.0)
    h = jnp.dot(o, w_ref[...], preferred_element_type=jnp.float32)
    h_ref[...] = h
    a_s = jnp.sum(h * asr_ref[...][None, :], axis=1)
    a_d = jnp.sum(h * adr_ref[...][None, :], axis=1)
    as_ref[...] = a_s
    ad_ref[...] = a_d
    t = jnp.max(a_s) + a_d
    m_ref[...] = jnp.where(t > 0, t, 0.2 * t)


def _tc_fin_body(op_ref, dp_ref, b_ref, wc_ref, bc_ref, y_ref):
    den = dp_ref[0, :] + dp_ref[1, :] + 1e-16
    o = (op_ref[0] + op_ref[1]) / den[:, None] + b_ref[...][None, :]
    o = jnp.maximum(o, 0.0)
    y_ref[...] = (jnp.dot(o, wc_ref[...], preferred_element_type=jnp.float32)
                  + bc_ref[...][None, :])


_f32 = jnp.float32

_tc_pre = pl.pallas_call(
    _tc_pre_body,
    out_shape=(jax.ShapeDtypeStruct((NP, CH), _f32),
               jax.ShapeDtypeStruct((NP,), _f32),
               jax.ShapeDtypeStruct((NP,), _f32),
               jax.ShapeDtypeStruct((NP,), _f32)),
)

_tc_mid = pl.pallas_call(
    _tc_mid_body,
    out_shape=(jax.ShapeDtypeStruct((NP, CH), _f32),
               jax.ShapeDtypeStruct((NP,), _f32),
               jax.ShapeDtypeStruct((NP,), _f32),
               jax.ShapeDtypeStruct((NP,), _f32)),
)

_tc_fin = pl.pallas_call(
    _tc_fin_body,
    out_shape=jax.ShapeDtypeStruct((NP, OUT), _f32),
)


def _sc_layer_body(h_hbm, as_hbm, ad_hbm, m_hbm, idx_hbm,
                   outp_hbm, denp_hbm,
                   rows_b, w_b, ase_b, ade_b, me_b, idx_b,
                   rows_sem, sc_sem, idx_sem, scat_sem,
                   out_sp, den_sp):
    c = lax.axis_index("c")
    s = lax.axis_index("s")
    base = jnp.where(c == 0, s * NB0, NS * NB0 + s * NB1)
    nb = jnp.where(c == 0, NB0, NB1)

    def _idx_copy(k):
        return pltpu.make_async_copy(
            idx_hbm.at[base + k], idx_b.at[lax.rem(k, IRING)], idx_sem)

    def _gather_descs(k, r):
        k8 = lax.rem(k, IRING)
        return [
            pltpu.make_async_copy(h_hbm.at[idx_b.at[k8, 0]], rows_b.at[r],
                                  rows_sem),
            pltpu.make_async_copy(as_hbm.at[idx_b.at[k8, 0]], ase_b.at[r],
                                  sc_sem),
            pltpu.make_async_copy(ad_hbm.at[idx_b.at[k8, 1]], ade_b.at[r],
                                  sc_sem),
            pltpu.make_async_copy(m_hbm.at[idx_b.at[k8, 1]], me_b.at[r],
                                  sc_sem),
        ]

    def _issue_scatters(k, r):
        k8 = lax.rem(k, IRING)
        pltpu.async_copy(w_b.at[r], den_sp.at[idx_b.at[k8, 1]], scat_sem,
                         add=True)
        pltpu.async_copy(rows_b.at[r], out_sp.at[idx_b.at[k8, 1]], scat_sem,
                         add=True)

    def _wait_scatters(k, r):
        k8 = lax.rem(k, IRING)
        pltpu.make_async_copy(w_b.at[r], den_sp.at[idx_b.at[k8, 1]],
                              scat_sem).wait()
        pltpu.make_async_copy(rows_b.at[r], out_sp.at[idx_b.at[k8, 1]],
                              scat_sem).wait()

    zeros16 = jnp.zeros((16,), _f32)

    @pl.loop(0, BLK)
    def _zero_rows(i):
        for cg in range(CH // 16):
            rows_b[0, i, pl.ds(cg * 16, 16)] = zeros16

    for g in range(BLK // 16):
        w_b[0, pl.ds(g * 16, 16)] = zeros16
    for t in range(RPT // BLK):
        pltpu.sync_copy(rows_b.at[0], out_sp.at[pl.ds(s * RPT + t * BLK, BLK)])
        pltpu.sync_copy(w_b.at[0], den_sp.at[pl.ds(s * RPT + t * BLK, BLK)])
    _rem = RPT - (RPT // BLK) * BLK
    if _rem:
        pltpu.sync_copy(rows_b.at[0].at[pl.ds(0, _rem)],
                        out_sp.at[pl.ds(s * RPT + (RPT // BLK) * BLK, _rem)])
        pltpu.sync_copy(w_b.at[0].at[pl.ds(0, _rem)],
                        den_sp.at[pl.ds(s * RPT + (RPT // BLK) * BLK, _rem)])
    plsc.subcore_barrier()

    def _step(k, r):
        for d in _gather_descs(k, r):
            d.wait()
        for g in range(BLK // 16):
            a_s = ase_b[r, pl.ds(g * 16, 16)]
            a_d = ade_b[r, pl.ds(g * 16, 16)]
            mm = me_b[r, pl.ds(g * 16, 16)]
            t = a_s + a_d
            e = jnp.where(t > 0, t, 0.2 * t)
            w_b[r, pl.ds(g * 16, 16)] = jnp.exp(e - mm)

        @pl.loop(0, BLK, unroll=4)
        def _scale(i):
            bidx = jnp.zeros((16,), jnp.int32) + i
            a16 = plsc.load_gather(w_b.at[r], [bidx])
            for cg in range(CH // 16):
                rows_b[r, i, pl.ds(cg * 16, 16)] = (
                    rows_b[r, i, pl.ds(cg * 16, 16)] * a16)

        _issue_scatters(k, r)

        @pl.when(k >= 1)
        def _drain():
            _wait_scatters(k - 1, (r - 1) % NRING)

        @pl.when(k + 2 < nb)
        def _prefetch():
            _idx_copy(k + 2).wait()
            for d in _gather_descs(k + 2, (r + 2) % NRING):
                d.start()

        @pl.when(k + 4 < nb)
        def _prefetch_idx():
            _idx_copy(k + 4).start()

    for k in range(4):
        _idx_copy(k).start()
    for k in range(2):
        _idx_copy(k).wait()
        for d in _gather_descs(k, k):
            d.start()

    @pl.loop(0, nb // NRING)
    def _outer(t):
        for b in range(NRING):
            _step(t * NRING + b, b)

    _wait_scatters(nb - 1, NRING - 1)

    plsc.subcore_barrier()
    pltpu.sync_copy(out_sp.at[pl.ds(s * RPT, RPT)],
                    outp_hbm.at[c, pl.ds(s * RPT, RPT)])
    pltpu.sync_copy(den_sp.at[pl.ds(s * RPT, RPT)],
                    denp_hbm.at[c, pl.ds(s * RPT, RPT)])


@functools.cache
def _make_sc_layer():
    mesh = plsc.VectorSubcoreMesh(core_axis_name="c", subcore_axis_name="s",
                                  num_cores=NC, num_subcores=NS)
    return pl.kernel(
        _sc_layer_body,
        out_type=(jax.ShapeDtypeStruct((NC, NP, CH), _f32),
                  jax.ShapeDtypeStruct((NC, NP), _f32)),
        mesh=mesh,
        compiler_params=pltpu.CompilerParams(needs_layout_passes=False),
        scratch_types=[
            pltpu.VMEM((NRING, BLK, CH), _f32),
            pltpu.VMEM((NRING, BLK), _f32),
            pltpu.VMEM((NRING, BLK), _f32),
            pltpu.VMEM((NRING, BLK), _f32),
            pltpu.VMEM((NRING, BLK), _f32),
            pltpu.VMEM((IRING, 2, BLK), jnp.int32),
            pltpu.SemaphoreType.DMA,
            pltpu.SemaphoreType.DMA,
            pltpu.SemaphoreType.DMA,
            pltpu.SemaphoreType.DMA,
            pltpu.VMEM_SHARED((NP, CH), _f32),
            pltpu.VMEM_SHARED((NP,), _f32),
        ],
    )


def kernel(x, edge_index, W0, a_src0, a_dst0, b0, W1, a_src1, a_dst1, b1,
           Wc, bc):
    ei = edge_index.astype(jnp.int32)
    ar = jnp.arange(N, dtype=jnp.int32)
    pad = jnp.full((EP - E_TOT,), PAD_IDX, jnp.int32)
    src = jnp.concatenate([ei[0], ar, pad]).reshape(TOTB, BLK)
    dst = jnp.concatenate([ei[1], ar, pad]).reshape(TOTB, BLK)
    idx = jnp.stack([src, dst], axis=1)
    xp = jnp.pad(x, ((0, NP - N), (0, 0)))

    sc_layer = _make_sc_layer()
    h0, as0, ad0, m0 = _tc_pre(xp, W0, a_src0, a_dst0)
    op0, dp0 = sc_layer(h0, as0, ad0, m0, idx)
    h1, as1, ad1, m1 = _tc_mid(op0, dp0, b0, W1, a_src1, a_dst1)
    op1, dp1 = sc_layer(h1, as1, ad1, m1, idx)
    y = _tc_fin(op1, dp1, b1, Wc, bc)
    return y[:N]


# speedup vs baseline: 1.1724x; 1.1724x over previous
import functools

import jax
import jax.numpy as jnp
from jax import lax
from jax.experimental import pallas as pl
from jax.experimental.pallas import tpu as pltpu
from jax.experimental.pallas import tpu_sc as plsc

N = 10000
NP = 10240
CH = 128
OUT = 64
E_RAW = 320000
E_TOT = E_RAW + N
NC = 2
NS = 16
NW = NC * NS
BLK = 112
NB0 = 114
NB1 = 72
TOTB = NS * (NB0 + NB1)
EP = TOTB * BLK
PAD_IDX = N
RPT = NP // NS
NRING = 3
IRING = 8


def _tc_pre_body(x_ref, w_ref, asr_ref, adr_ref, h_ref, as_ref, ad_ref, m_ref):
    h = jnp.dot(x_ref[...], w_ref[...], preferred_element_type=jnp.float32)
    h_ref[...] = h
    a_s = jnp.sum(h * asr_ref[...][None, :], axis=1)
    a_d = jnp.sum(h * adr_ref[...][None, :], axis=1)
    as_ref[...] = a_s
    ad_ref[...] = a_d
    t = jnp.max(a_s) + a_d
    m_ref[...] = jnp.where(t > 0, t, 0.2 * t)


def _tc_mid_body(op_ref, dp_ref, b_ref, w_ref, asr_ref, adr_ref,
                 h_ref, as_ref, ad_ref, m_ref):
    den = dp_ref[0, :] + dp_ref[1, :] + 1e-16
    o = (op_ref[0] + op_ref[1]) / den[:, None] + b_ref[...][None, :]
    o = jnp.maximum(o, 0.0)
    h = jnp.dot(o, w_ref[...], preferred_element_type=jnp.float32)
    h_ref[...] = h
    a_s = jnp.sum(h * asr_ref[...][None, :], axis=1)
    a_d = jnp.sum(h * adr_ref[...][None, :], axis=1)
    as_ref[...] = a_s
    ad_ref[...] = a_d
    t = jnp.max(a_s) + a_d
    m_ref[...] = jnp.where(t > 0, t, 0.2 * t)


def _tc_fin_body(op_ref, dp_ref, b_ref, wc_ref, bc_ref, y_ref):
    den = dp_ref[0, :] + dp_ref[1, :] + 1e-16
    o = (op_ref[0] + op_ref[1]) / den[:, None] + b_ref[...][None, :]
    o = jnp.maximum(o, 0.0)
    y_ref[...] = (jnp.dot(o, wc_ref[...], preferred_element_type=jnp.float32)
                  + bc_ref[...][None, :])


_f32 = jnp.float32

_tc_pre = pl.pallas_call(
    _tc_pre_body,
    out_shape=(jax.ShapeDtypeStruct((NP, CH), _f32),
               jax.ShapeDtypeStruct((NP,), _f32),
               jax.ShapeDtypeStruct((NP,), _f32),
               jax.ShapeDtypeStruct((NP,), _f32)),
)

_tc_mid = pl.pallas_call(
    _tc_mid_body,
    out_shape=(jax.ShapeDtypeStruct((NP, CH), _f32),
               jax.ShapeDtypeStruct((NP,), _f32),
               jax.ShapeDtypeStruct((NP,), _f32),
               jax.ShapeDtypeStruct((NP,), _f32)),
)

_tc_fin = pl.pallas_call(
    _tc_fin_body,
    out_shape=jax.ShapeDtypeStruct((NP, OUT), _f32),
)


def _sc_layer_body(h_hbm, as_hbm, ad_hbm, m_hbm, idx_hbm,
                   outp_hbm, denp_hbm,
                   rows_b, w_b, ase_b, ade_b, me_b, idx_b,
                   rows_sem, sc_sem, idx_sem, scat_sem,
                   out_sp, den_sp):
    c = lax.axis_index("c")
    s = lax.axis_index("s")
    base = jnp.where(c == 0, s * NB0, NS * NB0 + s * NB1)
    nb = jnp.where(c == 0, NB0, NB1)

    def _idx_copy(k):
        return pltpu.make_async_copy(
            idx_hbm.at[base + k], idx_b.at[lax.rem(k, IRING)], idx_sem)

    def _gather_descs(k, r):
        k8 = lax.rem(k, IRING)
        return [
            pltpu.make_async_copy(h_hbm.at[idx_b.at[k8, 0]], rows_b.at[r],
                                  rows_sem),
            pltpu.make_async_copy(as_hbm.at[idx_b.at[k8, 0]], ase_b.at[r],
                                  sc_sem),
            pltpu.make_async_copy(ad_hbm.at[idx_b.at[k8, 1]], ade_b.at[r],
                                  sc_sem),
            pltpu.make_async_copy(m_hbm.at[idx_b.at[k8, 1]], me_b.at[r],
                                  sc_sem),
        ]

    def _issue_scatters(k, r):
        k8 = lax.rem(k, IRING)
        pltpu.async_copy(w_b.at[r], den_sp.at[idx_b.at[k8, 1]], scat_sem,
                         add=True)
        pltpu.async_copy(rows_b.at[r], out_sp.at[idx_b.at[k8, 1]], scat_sem,
                         add=True)

    def _wait_scatters(k, r):
        k8 = lax.rem(k, IRING)
        pltpu.make_async_copy(w_b.at[r], den_sp.at[idx_b.at[k8, 1]],
                              scat_sem).wait()
        pltpu.make_async_copy(rows_b.at[r], out_sp.at[idx_b.at[k8, 1]],
                              scat_sem).wait()

    zeros16 = jnp.zeros((16,), _f32)

    @pl.loop(0, BLK)
    def _zero_rows(i):
        for cg in range(CH // 16):
            rows_b[0, i, pl.ds(cg * 16, 16)] = zeros16

    for g in range(BLK // 16):
        w_b[0, pl.ds(g * 16, 16)] = zeros16
    for t in range(RPT // BLK):
        pltpu.sync_copy(rows_b.at[0], out_sp.at[pl.ds(s * RPT + t * BLK, BLK)])
        pltpu.sync_copy(w_b.at[0], den_sp.at[pl.ds(s * RPT + t * BLK, BLK)])
    _rem = RPT - (RPT // BLK) * BLK
    if _rem:
        pltpu.sync_copy(rows_b.at[0].at[pl.ds(0, _rem)],
                        out_sp.at[pl.ds(s * RPT + (RPT // BLK) * BLK, _rem)])
        pltpu.sync_copy(w_b.at[0].at[pl.ds(0, _rem)],
                        den_sp.at[pl.ds(s * RPT + (RPT // BLK) * BLK, _rem)])
    plsc.subcore_barrier()

    def _step(k, r):
        for d in _gather_descs(k, r):
            d.wait()
        for g in range(BLK // 16):
            a_s = ase_b[r, pl.ds(g * 16, 16)]
            a_d = ade_b[r, pl.ds(g * 16, 16)]
            mm = me_b[r, pl.ds(g * 16, 16)]
            t = a_s + a_d
            e = jnp.where(t > 0, t, 0.2 * t)
            w_b[r, pl.ds(g * 16, 16)] = jnp.exp(e - mm)

        @pl.loop(0, BLK, unroll=4)
        def _scale(i):
            bidx = jnp.zeros((16,), jnp.int32) + i
            a16 = plsc.load_gather(w_b.at[r], [bidx])
            for cg in range(CH // 16):
                rows_b[r, i, pl.ds(cg * 16, 16)] = (
                    rows_b[r, i, pl.ds(cg * 16, 16)] * a16)

        _issue_scatters(k, r)

        @pl.when(k >= 1)
        def _drain():
            _wait_scatters(k - 1, (r - 1) % NRING)

        @pl.when(k + 2 < nb)
        def _prefetch():
            _idx_copy(k + 2).wait()
            for d in _gather_descs(k + 2, (r + 2) % NRING):
                d.start()

        @pl.when(k + 4 < nb)
        def _prefetch_idx():
            _idx_copy(k + 4).start()

    for k in range(4):
        _idx_copy(k).start()
    for k in range(2):
        _idx_copy(k).wait()
        for d in _gather_descs(k, k):
            d.start()

    @pl.loop(0, nb // NRING)
    def _outer(t):
        for b in range(NRING):
            _step(t * NRING + b, b)

    _wait_scatters(nb - 1, NRING - 1)

    plsc.subcore_barrier()
    pltpu.sync_copy(out_sp.at[pl.ds(s * RPT, RPT)],
                    outp_hbm.at[c, pl.ds(s * RPT, RPT)])
    pltpu.sync_copy(den_sp.at[pl.ds(s * RPT, RPT)],
                    denp_hbm.at[c, pl.ds(s * RPT, RPT)])


@functools.cache
def _make_sc_layer():
    mesh = plsc.VectorSubcoreMesh(core_axis_name="c", subcore_axis_name="s",
                                  num_cores=NC, num_subcores=NS)
    return pl.kernel(
        _sc_layer_body,
        out_type=(jax.ShapeDtypeStruct((NC, NP, CH), _f32),
                  jax.ShapeDtypeStruct((NC, NP), _f32)),
        mesh=mesh,
        compiler_params=pltpu.CompilerParams(needs_layout_passes=False),
        scratch_types=[
            pltpu.VMEM((NRING, BLK, CH), _f32),
            pltpu.VMEM((NRING, BLK), _f32),
            pltpu.VMEM((NRING, BLK), _f32),
            pltpu.VMEM((NRING, BLK), _f32),
            pltpu.VMEM((NRING, BLK), _f32),
            pltpu.VMEM((IRING, 2, BLK), jnp.int32),
            pltpu.SemaphoreType.DMA,
            pltpu.SemaphoreType.DMA,
            pltpu.SemaphoreType.DMA,
            pltpu.SemaphoreType.DMA,
            pltpu.VMEM_SHARED((NP, CH), _f32),
            pltpu.VMEM_SHARED((NP,), _f32),
        ],
    )


def kernel(x, edge_index, W0, a_src0, a_dst0, b0, W1, a_src1, a_dst1, b1,
           Wc, bc):
    ei = edge_index.astype(jnp.int32)
    ar = jnp.arange(N, dtype=jnp.int32)
    pad = jnp.full((EP - E_TOT,), PAD_IDX, jnp.int32)
    src = jnp.concatenate([ei[0], ar, pad]).reshape(TOTB, BLK)
    dst = jnp.concatenate([ei[1], ar, pad]).reshape(TOTB, BLK)
    idx = jnp.stack([src, dst], axis=1)
    xp = jnp.pad(x, ((0, NP - N), (0, 0)))

    sc_layer = _make_sc_layer()
    h0, as0, ad0, m0 = _tc_pre(xp, W0, a_src0, a_dst0)
    op0, dp0 = sc_layer(h0, as0, ad0, m0, idx)
    h1, as1, ad1, m1 = _tc_mid(op0, dp0, b0, W1, a_src1, a_dst1)
    op1, dp1 = sc_layer(h1, as1, ad1, m1, idx)
    y = _tc_fin(op1, dp1, b1, Wc, bc)
    return y[:N]


# speedup vs baseline: 1.1726x; 1.0002x over previous
import functools

import jax
import jax.numpy as jnp
from jax import lax
from jax.experimental import pallas as pl
from jax.experimental.pallas import tpu as pltpu
from jax.experimental.pallas import tpu_sc as plsc

N = 10000
NP = 10112
CH = 128
OUT = 64
E_RAW = 320000
E_TOT = E_RAW + N
NC = 2
NS = 16
NW = NC * NS
BLK = 112
NB0 = 114
NB1 = 72
TOTB = NS * (NB0 + NB1)
EP = TOTB * BLK
PAD_IDX = N
RPT = NP // NS
RNZ = 640
RNT = NP - (NS - 1) * RNZ
NRING = 3
IRING = 6


def _tc_pre_body(x_ref, w_ref, asr_ref, adr_ref, h_ref, as_ref, ad_ref, m_ref):
    h = jnp.dot(x_ref[...], w_ref[...], preferred_element_type=jnp.float32)
    h_ref[...] = h
    a_s = jnp.sum(h * asr_ref[...][None, :], axis=1)
    a_d = jnp.sum(h * adr_ref[...][None, :], axis=1)
    as_ref[...] = a_s
    ad_ref[...] = a_d
    t = jnp.max(a_s) + a_d
    m_ref[...] = jnp.where(t > 0, t, 0.2 * t)


def _tc_mid_body(op_ref, dp_ref, b_ref, w_ref, asr_ref, adr_ref,
                 h_ref, as_ref, ad_ref, m_ref):
    dp = dp_ref[...]
    den = dp[:NP] + dp[NP:] + 1e-16
    o = (op_ref[0] + op_ref[1]) / den[:, None] + b_ref[...][None, :]
    o = jnp.maximum(o, 0.0)
    h = jnp.dot(o, w_ref[...], preferred_element_type=jnp.float32)
    h_ref[...] = h
    a_s = jnp.sum(h * asr_ref[...][None, :], axis=1)
    a_d = jnp.sum(h * adr_ref[...][None, :], axis=1)
    as_ref[...] = a_s
    ad_ref[...] = a_d
    t = jnp.max(a_s) + a_d
    m_ref[...] = jnp.where(t > 0, t, 0.2 * t)


def _tc_fin_body(op_ref, dp_ref, b_ref, wc_ref, bc_ref, y_ref):
    dp = dp_ref[...]
    den = dp[:NP] + dp[NP:] + 1e-16
    o = (op_ref[0] + op_ref[1]) / den[:, None] + b_ref[...][None, :]
    o = jnp.maximum(o, 0.0)
    y_ref[...] = (jnp.dot(o, wc_ref[...], preferred_element_type=jnp.float32)
                  + bc_ref[...][None, :])


_f32 = jnp.float32

_tc_pre = pl.pallas_call(
    _tc_pre_body,
    out_shape=(jax.ShapeDtypeStruct((NP, CH), _f32),
               jax.ShapeDtypeStruct((NP,), _f32),
               jax.ShapeDtypeStruct((NP,), _f32),
               jax.ShapeDtypeStruct((NP,), _f32)),
)

_tc_mid = pl.pallas_call(
    _tc_mid_body,
    out_shape=(jax.ShapeDtypeStruct((NP, CH), _f32),
               jax.ShapeDtypeStruct((NP,), _f32),
               jax.ShapeDtypeStruct((NP,), _f32),
               jax.ShapeDtypeStruct((NP,), _f32)),
)

_tc_fin = pl.pallas_call(
    _tc_fin_body,
    out_shape=jax.ShapeDtypeStruct((NP, OUT), _f32),
)


def _sc_layer_body(h_hbm, as_hbm, ad_hbm, m_hbm, idx_hbm,
                   outp_hbm, denp_hbm,
                   rows_b, w_b, ase_b, ade_b, me_b, idx_b,
                   rows_sem, sc_sem, idx_sem, scat_sem,
                   bounce_v, out_sp, den_sp, as_sp, ad_sp, m_sp):
    c = lax.axis_index("c")
    s = lax.axis_index("s")
    base = jnp.where(c == 0, s * NB0, NS * NB0 + s * NB1)
    nb = jnp.where(c == 0, NB0, NB1)

    def _idx_copy(k):
        return pltpu.make_async_copy(
            idx_hbm.at[base + k], idx_b.at[lax.rem(k, IRING)], idx_sem)

    def _gather_descs(k, r):
        k8 = lax.rem(k, IRING)
        return [
            pltpu.make_async_copy(h_hbm.at[idx_b.at[k8, 0]], rows_b.at[r],
                                  rows_sem),
            pltpu.make_async_copy(as_sp.at[idx_b.at[k8, 0]], ase_b.at[r],
                                  sc_sem),
            pltpu.make_async_copy(ad_sp.at[idx_b.at[k8, 1]], ade_b.at[r],
                                  sc_sem),
            pltpu.make_async_copy(m_sp.at[idx_b.at[k8, 1]], me_b.at[r],
                                  sc_sem),
        ]

    def _issue_scatters(k, r):
        k8 = lax.rem(k, IRING)
        pltpu.async_copy(w_b.at[r], den_sp.at[idx_b.at[k8, 1]], scat_sem,
                         add=True)
        pltpu.async_copy(rows_b.at[r], out_sp.at[idx_b.at[k8, 1]], scat_sem,
                         add=True)

    def _wait_scatters(k, r):
        k8 = lax.rem(k, IRING)
        pltpu.make_async_copy(w_b.at[r], den_sp.at[idx_b.at[k8, 1]],
                              scat_sem).wait()
        pltpu.make_async_copy(rows_b.at[r], out_sp.at[idx_b.at[k8, 1]],
                              scat_sem).wait()

    zeros16 = jnp.zeros((16,), _f32)

    @pl.loop(0, BLK)
    def _zero_rows(i):
        for cg in range(CH // 16):
            rows_b[0, i, pl.ds(cg * 16, 16)] = zeros16

    @pl.loop(0, RNZ // 16)
    def _zero_bounce(g):
        bounce_v[pl.ds(g * 16, 16)] = zeros16

    for t in range(RPT // BLK):
        pltpu.sync_copy(rows_b.at[0], out_sp.at[pl.ds(s * RPT + t * BLK, BLK)])
    _rem = RPT - (RPT // BLK) * BLK
    if _rem:
        pltpu.sync_copy(rows_b.at[0].at[pl.ds(0, _rem)],
                        out_sp.at[pl.ds(s * RPT + (RPT // BLK) * BLK, _rem)])

    def _stage(n):
        pltpu.sync_copy(bounce_v.at[pl.ds(0, n)],
                        den_sp.at[pl.ds(s * RNZ, n)])
        for src_hbm, dst_sp in ((as_hbm, as_sp), (ad_hbm, ad_sp),
                                (m_hbm, m_sp)):
            pltpu.sync_copy(src_hbm.at[pl.ds(s * RNZ, n)],
                            bounce_v.at[pl.ds(0, n)])
            pltpu.sync_copy(bounce_v.at[pl.ds(0, n)],
                            dst_sp.at[pl.ds(s * RNZ, n)])

    @pl.when(s < NS - 1)
    def _stage_full():
        _stage(RNZ)

    @pl.when(s == NS - 1)
    def _stage_tail():
        _stage(RNT)

    plsc.subcore_barrier()

    def _step(k, r):
        for d in _gather_descs(k, r):
            d.wait()
        for g in range(BLK // 16):
            a_s = ase_b[r, pl.ds(g * 16, 16)]
            a_d = ade_b[r, pl.ds(g * 16, 16)]
            mm = me_b[r, pl.ds(g * 16, 16)]
            t = a_s + a_d
            e = jnp.where(t > 0, t, 0.2 * t)
            w_b[r, pl.ds(g * 16, 16)] = jnp.exp(e - mm)

        @pl.loop(0, BLK, unroll=4)
        def _scale(i):
            bidx = jnp.zeros((16,), jnp.int32) + i
            a16 = plsc.load_gather(w_b.at[r], [bidx])
            for cg in range(CH // 16):
                rows_b[r, i, pl.ds(cg * 16, 16)] = (
                    rows_b[r, i, pl.ds(cg * 16, 16)] * a16)

        _issue_scatters(k, r)

        @pl.when(k >= 1)
        def _drain():
            _wait_scatters(k - 1, (r - 1) % NRING)

        @pl.when(k + 2 < nb)
        def _prefetch():
            _idx_copy(k + 2).wait()
            for d in _gather_descs(k + 2, (r + 2) % NRING):
                d.start()

        @pl.when(k + 4 < nb)
        def _prefetch_idx():
            _idx_copy(k + 4).start()

    for k in range(4):
        _idx_copy(k).start()
    for k in range(2):
        _idx_copy(k).wait()
        for d in _gather_descs(k, k):
            d.start()

    @pl.loop(0, nb // NRING)
    def _outer(t):
        for b in range(NRING):
            _step(t * NRING + b, b)

    _wait_scatters(nb - 1, NRING - 1)

    plsc.subcore_barrier()
    pltpu.sync_copy(out_sp.at[pl.ds(s * RPT, RPT)],
                    outp_hbm.at[c, pl.ds(s * RPT, RPT)])

    @pl.when(s < NS - 1)
    def _den_out_full():
        pltpu.sync_copy(den_sp.at[pl.ds(s * RNZ, RNZ)],
                        denp_hbm.at[pl.ds(c * NP + s * RNZ, RNZ)])

    @pl.when(s == NS - 1)
    def _den_out_tail():
        pltpu.sync_copy(den_sp.at[pl.ds(s * RNZ, RNT)],
                        denp_hbm.at[pl.ds(c * NP + s * RNZ, RNT)])


@functools.cache
def _make_sc_layer():
    mesh = plsc.VectorSubcoreMesh(core_axis_name="c", subcore_axis_name="s",
                                  num_cores=NC, num_subcores=NS)
    return pl.kernel(
        _sc_layer_body,
        out_type=(jax.ShapeDtypeStruct((NC, NP, CH), _f32),
                  jax.ShapeDtypeStruct((NC * NP,), _f32)),
        mesh=mesh,
        compiler_params=pltpu.CompilerParams(needs_layout_passes=False),
        scratch_types=[
            pltpu.VMEM((NRING, BLK, CH), _f32),
            pltpu.VMEM((NRING, BLK), _f32),
            pltpu.VMEM((NRING, BLK), _f32),
            pltpu.VMEM((NRING, BLK), _f32),
            pltpu.VMEM((NRING, BLK), _f32),
            pltpu.VMEM((IRING, 2, BLK), jnp.int32),
            pltpu.SemaphoreType.DMA,
            pltpu.SemaphoreType.DMA,
            pltpu.SemaphoreType.DMA,
            pltpu.SemaphoreType.DMA,
            pltpu.VMEM((RNZ,), _f32),
            pltpu.VMEM_SHARED((NP, CH), _f32),
            pltpu.VMEM_SHARED((NP,), _f32),
            pltpu.VMEM_SHARED((NP,), _f32),
            pltpu.VMEM_SHARED((NP,), _f32),
            pltpu.VMEM_SHARED((NP,), _f32),
        ],
    )


def kernel(x, edge_index, W0, a_src0, a_dst0, b0, W1, a_src1, a_dst1, b1,
           Wc, bc):
    ei = edge_index.astype(jnp.int32)
    ar = jnp.arange(N, dtype=jnp.int32)
    pad = jnp.full((EP - E_TOT,), PAD_IDX, jnp.int32)
    src = jnp.concatenate([ei[0], ar, pad]).reshape(TOTB, BLK)
    dst = jnp.concatenate([ei[1], ar, pad]).reshape(TOTB, BLK)
    idx = jnp.stack([src, dst], axis=1)
    xp = jnp.pad(x, ((0, NP - N), (0, 0)))

    sc_layer = _make_sc_layer()
    h0, as0, ad0, m0 = _tc_pre(xp, W0, a_src0, a_dst0)
    op0, dp0 = sc_layer(h0, as0, ad0, m0, idx)
    h1, as1, ad1, m1 = _tc_mid(op0, dp0, b0, W1, a_src1, a_dst1)
    op1, dp1 = sc_layer(h1, as1, ad1, m1, idx)
    y = _tc_fin(op1, dp1, b1, Wc, bc)
    return y[:N]
